# baseline (speedup 1.0000x reference)
import functools

import jax
import jax.numpy as jnp
from jax import lax
from jax.experimental import pallas as pl
from jax.experimental.pallas import tpu as pltpu
from jax.experimental.pallas import tpu_sc as plsc

B, C, N = 8, 64, 65536
K_TOP = 16384
CHUNK = 8192
NCHUNK = N // CHUNK
ROWS, LANES = 512, 128


def _maxkey_body(y_ref, out_ref):
    yv = y_ref[0]
    z = jnp.max(yv, axis=0, keepdims=True)
    bits = jax.lax.bitcast_convert_type(z, jnp.int32)
    key = jnp.where(bits >= 0, ~bits, bits ^ jnp.int32(-2147483648))
    out_ref[0] = key


def _maxkey(y):
    return pl.pallas_call(
        _maxkey_body,
        grid=(B, NCHUNK),
        in_specs=[pl.BlockSpec((1, C, CHUNK), lambda b, c: (b, 0, c))],
        out_specs=pl.BlockSpec((1, 1, CHUNK), lambda b, c: (b * NCHUNK + c, 0, 0)),
        out_shape=jax.ShapeDtypeStruct((B * NCHUNK, 1, CHUNK), jnp.int32),
    )(y)


def _rot(a, sh, axis):
    n = a.shape[axis]
    sh = sh % n
    if sh == 0:
        return a
    idx_hi = [slice(None)] * a.ndim
    idx_lo = [slice(None)] * a.ndim
    idx_hi[axis] = slice(sh, None)
    idx_lo[axis] = slice(None, sh)
    return jnp.concatenate([a[tuple(idx_hi)], a[tuple(idx_lo)]], axis=axis)


CHROWS = 128


def _lex_bLTa(Ka, Ia, Kb, Ib):
    return (Kb < Ka) | ((Kb == Ka) & (Ib < Ia))


def _pair_rows(arr, s):
    sh = arr.shape
    g = sh[1] // (2 * s)
    v = arr.reshape(sh[0], g, 2, s, sh[2])
    return v[:, :, 0], v[:, :, 1]


def _unpair_rows(a, b):
    sh = a.shape
    v = jnp.concatenate([a[:, :, None], b[:, :, None]], axis=2)
    return v.reshape(sh[0], sh[1] * 2 * sh[2], sh[3])


def _sublane_stage(K, I, dirA, s):
    Ka, Kb = _pair_rows(K, s)
    Ia, Ib = _pair_rows(I, s)
    da, _ = _pair_rows(jnp.broadcast_to(dirA, K.shape), s)
    swap = _lex_bLTa(Ka, Ia, Kb, Ib) == da
    nKa = jnp.where(swap, Kb, Ka)
    nKb = jnp.where(swap, Ka, Kb)
    nIa = jnp.where(swap, Ib, Ia)
    nIb = jnp.where(swap, Ia, Ib)
    return _unpair_rows(nKa, nKb), _unpair_rows(nIa, nIb)


def _rot(a, sh, axis):
    n = a.shape[axis]
    return pltpu.roll(a, (-sh) % n, axis)


def _roll_stage(K, I, dirA, st, axis):
    iot = lax.broadcasted_iota(jnp.int32, K.shape[1:], axis - 1)[None]
    low = (iot & st) == 0
    Kp = jnp.where(low, _rot(K, st, axis), _rot(K, -st, axis))
    Ip = jnp.where(low, _rot(I, st, axis), _rot(I, -st, axis))
    pLTm = _lex_bLTa(K, I, Kp, Ip)
    takeP = (low == dirA) == pLTm
    return jnp.where(takeP, Kp, K), jnp.where(takeP, Ip, I)


def _stage(K, I, dirA, s):
    if 8 <= s <= 64:
        return _roll_stage(K, I, dirA, s, 1)
    if s <= 4:
        return _roll_stage(K, I, dirA, s, 1)
    return _roll_stage(K, I, dirA, s // CHROWS, 2)


def _prune_pairs(K, I):
    sh = K.shape
    g = sh[1] // (2 * CHROWS)
    Kv = K.reshape(sh[0], g, 2, CHROWS, sh[2])
    Iv = I.reshape(sh[0], g, 2, CHROWS, sh[2])
    Ka, Kb = Kv[:, :, 0], Kv[:, :, 1]
    Ia, Ib = Iv[:, :, 0], Iv[:, :, 1]
    t = _lex_bLTa(Ka, Ia, Kb, Ib)
    nK = jnp.where(t, Kb, Ka).reshape(sh[0], g * CHROWS, sh[2])
    nI = jnp.where(t, Ib, Ia).reshape(sh[0], g * CHROWS, sh[2])
    return nK, nI


def _sort_body(k_ref, out_ref):
    K = k_ref[...]
    Pr = lax.broadcasted_iota(jnp.int32, (1, ROWS, LANES), 1)
    Pc = lax.broadcasted_iota(jnp.int32, (1, ROWS, LANES), 2)
    I = jnp.broadcast_to(Pr * LANES + Pc, K.shape)
    J = (Pr >> 7) * K_TOP + Pc * CHROWS + (Pr & 127)

    m = 2
    while m <= K_TOP:
        dirA = (J & m) == 0
        s = m // 2
        while s >= 1:
            K, I = _stage(K, I, dirA, s)
            s //= 2
        m *= 2

    K, I = _prune_pairs(K, I)
    dirA = lax.broadcasted_iota(jnp.int32, (1, 2 * CHROWS, 1), 1) < CHROWS
    s = K_TOP // 2
    while s >= 1:
        K, I = _stage(K, I, dirA, s)
        s //= 2
    K, I = _prune_pairs(K, I)
    dirA = jnp.full((1, 1, 1), True)
    s = K_TOP // 2
    while s >= 1:
        K, I = _stage(K, I, dirA, s)
        s //= 2
    out_ref[...] = I


def _sort(keys):
    return pl.pallas_call(
        _sort_body,
        in_specs=[pl.BlockSpec((B, ROWS, LANES), lambda: (0, 0, 0))],
        out_specs=pl.BlockSpec((B, CHROWS, LANES), lambda: (0, 0, 0)),
        out_shape=jax.ShapeDtypeStruct((B, CHROWS, LANES), jnp.int32),
    )(keys)


CX = 3


HALF = K_TOP // 2


def _gather_half(row_v, idx_v, out_v, h):
    @plsc.parallel_loop(h * HALF, (h + 1) * HALF, 16, unroll=16)
    def _(i):
        iv = idx_v[pl.ds(i, 16)]
        out_v[pl.ds(i - h * HALF, 16)] = plsc.load_gather(row_v, [iv])


def _sc_gather_body(
    x_hbm, y_hbm, idx_hbm, out_x, out_y, row_v, idx_v, oa_v, ob_v, sem_a, sem_b
):
    cid = lax.axis_index("c")
    sid = lax.axis_index("s")
    wid = sid * 2 + cid
    b = wid // 4
    j = wid % 4
    pltpu.sync_copy(idx_hbm.at[b], idx_v)

    def do_row(src_row, dst_row, prev):
        pltpu.sync_copy(src_row, row_v)
        if prev is not None:
            prev[0].wait()
            prev[1].wait()
        _gather_half(row_v, idx_v, oa_v, 0)
        da = pltpu.async_copy(oa_v, dst_row.at[pl.ds(0, HALF)], sem_a)
        _gather_half(row_v, idx_v, ob_v, 1)
        db = pltpu.async_copy(ob_v, dst_row.at[pl.ds(HALF, HALF)], sem_b)
        return (da, db)

    prev = None
    for ci in range(C // 4):
        c = j * (C // 4) + ci
        prev = do_row(y_hbm.at[b, c], out_y.at[b, c], prev)
    prev[0].wait()
    prev[1].wait()

    @pl.when(j < CX)
    def _():
        pltpu.sync_copy(x_hbm.at[b, j], row_v)
        _gather_half(row_v, idx_v, oa_v, 0)
        _gather_half(row_v, idx_v, ob_v, 1)
        pltpu.sync_copy(oa_v, out_x.at[b, j, pl.ds(0, HALF)])
        pltpu.sync_copy(ob_v, out_x.at[b, j, pl.ds(HALF, HALF)])


@functools.cache
def _make_sc_gather():
    return pl.kernel(
        _sc_gather_body,
        out_type=(
            jax.ShapeDtypeStruct((B, CX, K_TOP), jnp.float32),
            jax.ShapeDtypeStruct((B, C, K_TOP), jnp.float32),
        ),
        mesh=plsc.VectorSubcoreMesh(core_axis_name="c", subcore_axis_name="s"),
        compiler_params=pltpu.CompilerParams(needs_layout_passes=False),
        scratch_types=[
            pltpu.VMEM((N,), jnp.float32),
            pltpu.VMEM((K_TOP,), jnp.int32),
            pltpu.VMEM((HALF,), jnp.float32),
            pltpu.VMEM((HALF,), jnp.float32),
            pltpu.SemaphoreType.DMA,
            pltpu.SemaphoreType.DMA,
        ],
    )


def kernel(x, y):
    keys = _maxkey(y)
    keys = keys.reshape(B, ROWS, LANES)
    idx = jnp.swapaxes(_sort(keys), 1, 2).reshape(B, K_TOP)
    top_k_xyz, top_k_points = _make_sc_gather()(x, y, idx)
    return (top_k_xyz, top_k_points)


# speedup vs baseline: 1.1497x; 1.1497x over previous
import functools

import jax
import jax.numpy as jnp
from jax import lax
from jax.experimental import pallas as pl
from jax.experimental.pallas import tpu as pltpu
from jax.experimental.pallas import tpu_sc as plsc

B, C, N = 8, 64, 65536
K_TOP = 16384
CHUNK = 8192
NCHUNK = N // CHUNK
ROWS, LANES = 512, 128


def _maxkey_body(y_ref, out_ref):
    yv = y_ref[0]
    z = jnp.max(yv, axis=0, keepdims=True)
    bits = jax.lax.bitcast_convert_type(z, jnp.int32)
    key = jnp.where(bits >= 0, ~bits, bits ^ jnp.int32(-2147483648))
    out_ref[0] = key


def _maxkey(y):
    return pl.pallas_call(
        _maxkey_body,
        grid=(B, NCHUNK),
        in_specs=[pl.BlockSpec((1, C, CHUNK), lambda b, c: (b, 0, c))],
        out_specs=pl.BlockSpec((1, 1, CHUNK), lambda b, c: (b * NCHUNK + c, 0, 0)),
        out_shape=jax.ShapeDtypeStruct((B * NCHUNK, 1, CHUNK), jnp.int32),
    )(y)


def _rot(a, sh, axis):
    n = a.shape[axis]
    sh = sh % n
    if sh == 0:
        return a
    idx_hi = [slice(None)] * a.ndim
    idx_lo = [slice(None)] * a.ndim
    idx_hi[axis] = slice(sh, None)
    idx_lo[axis] = slice(None, sh)
    return jnp.concatenate([a[tuple(idx_hi)], a[tuple(idx_lo)]], axis=axis)


CHROWS = 128
_TR_MIN_M = 2048


def _lex_bLTa(Ka, Ia, Kb, Ib):
    return (Kb < Ka) | ((Kb == Ka) & (Ib < Ia))


def _pair_rows(arr, s):
    sh = arr.shape
    g = sh[1] // (2 * s)
    v = arr.reshape(sh[0], g, 2, s, sh[2])
    return v[:, :, 0], v[:, :, 1]


def _unpair_rows(a, b):
    sh = a.shape
    v = jnp.concatenate([a[:, :, None], b[:, :, None]], axis=2)
    return v.reshape(sh[0], sh[1] * 2 * sh[2], sh[3])


def _sublane_stage(K, I, dirA, s):
    Ka, Kb = _pair_rows(K, s)
    Ia, Ib = _pair_rows(I, s)
    da, _ = _pair_rows(jnp.broadcast_to(dirA, K.shape), s)
    swap = _lex_bLTa(Ka, Ia, Kb, Ib) == da
    nKa = jnp.where(swap, Kb, Ka)
    nKb = jnp.where(swap, Ka, Kb)
    nIa = jnp.where(swap, Ib, Ia)
    nIb = jnp.where(swap, Ia, Ib)
    return _unpair_rows(nKa, nKb), _unpair_rows(nIa, nIb)


def _rot(a, sh, axis):
    n = a.shape[axis]
    return pltpu.roll(a, (-sh) % n, axis)


def _roll_stage(K, I, dirA, st, axis):
    iot = lax.broadcasted_iota(jnp.int32, K.shape[1:], axis - 1)[None]
    low = (iot & st) == 0
    Kp = jnp.where(low, _rot(K, st, axis), _rot(K, -st, axis))
    Ip = jnp.where(low, _rot(I, st, axis), _rot(I, -st, axis))
    pLTm = _lex_bLTa(K, I, Kp, Ip)
    takeP = (low == dirA) == pLTm
    return jnp.where(takeP, Kp, K), jnp.where(takeP, Ip, I)


def _stage(K, I, dirA, s):
    if 8 <= s <= 64:
        return _sublane_stage(K, I, dirA, s)
    if s <= 4:
        return _roll_stage(K, I, dirA, s, 1)
    return _roll_stage(K, I, dirA, s // CHROWS, 2)


def _tr(a):
    sh = a.shape
    v = a.reshape(sh[0], sh[1] // CHROWS, CHROWS, LANES)
    return jnp.swapaxes(v, 2, 3).reshape(sh)


def _merge_tail(K, I, dirA, dirT, m):
    KT, IT = _tr(K), _tr(I)
    sl = m // (2 * CHROWS)
    while sl >= 1:
        KT, IT = _stage(KT, IT, dirT, sl)
        sl //= 2
    K, I = _tr(KT), _tr(IT)
    s = 64
    while s >= 1:
        K, I = _stage(K, I, dirA, s)
        s //= 2
    return K, I


def _prune_pairs(K, I):
    sh = K.shape
    g = sh[1] // (2 * CHROWS)
    Kv = K.reshape(sh[0], g, 2, CHROWS, sh[2])
    Iv = I.reshape(sh[0], g, 2, CHROWS, sh[2])
    Ka, Kb = Kv[:, :, 0], Kv[:, :, 1]
    Ia, Ib = Iv[:, :, 0], Iv[:, :, 1]
    t = _lex_bLTa(Ka, Ia, Kb, Ib)
    nK = jnp.where(t, Kb, Ka).reshape(sh[0], g * CHROWS, sh[2])
    nI = jnp.where(t, Ib, Ia).reshape(sh[0], g * CHROWS, sh[2])
    return nK, nI


def _sort_body(k_ref, out_ref):
    K = k_ref[...]
    Pr = lax.broadcasted_iota(jnp.int32, (1, ROWS, LANES), 1)
    Pc = lax.broadcasted_iota(jnp.int32, (1, ROWS, LANES), 2)
    I = jnp.broadcast_to(Pr * LANES + Pc, K.shape)
    J = (Pr >> 7) * K_TOP + Pc * CHROWS + (Pr & 127)
    JT = (Pr >> 7) * K_TOP + (Pr & 127) * CHROWS + Pc

    m = 2
    while m < _TR_MIN_M:
        dirA = (J & m) == 0
        s = m // 2
        while s >= 1:
            K, I = _stage(K, I, dirA, s)
            s //= 2
        m *= 2
    while m <= K_TOP:
        K, I = _merge_tail(K, I, (J & m) == 0, (JT & m) == 0, m)
        m *= 2

    K, I = _prune_pairs(K, I)
    dirA = lax.broadcasted_iota(jnp.int32, (1, 2 * CHROWS, 1), 1) < CHROWS
    K, I = _merge_tail(K, I, dirA, dirA, K_TOP)
    K, I = _prune_pairs(K, I)
    dirA = jnp.full((1, 1, 1), True)
    K, I = _merge_tail(K, I, dirA, dirA, K_TOP)
    out_ref[...] = I


def _sort(keys):
    return pl.pallas_call(
        _sort_body,
        in_specs=[pl.BlockSpec((B, ROWS, LANES), lambda: (0, 0, 0))],
        out_specs=pl.BlockSpec((B, CHROWS, LANES), lambda: (0, 0, 0)),
        out_shape=jax.ShapeDtypeStruct((B, CHROWS, LANES), jnp.int32),
    )(keys)


CX = 3


HALF = K_TOP // 2


def _gather_half(row_v, idx_v, out_v, h):
    @plsc.parallel_loop(h * HALF, (h + 1) * HALF, 16, unroll=16)
    def _(i):
        iv = idx_v[pl.ds(i, 16)]
        out_v[pl.ds(i - h * HALF, 16)] = plsc.load_gather(row_v, [iv])


def _sc_gather_body(
    x_hbm, y_hbm, idx_hbm, out_x, out_y, row_v, idx_v, oa_v, ob_v, sem_a, sem_b
):
    cid = lax.axis_index("c")
    sid = lax.axis_index("s")
    wid = sid * 2 + cid
    b = wid // 4
    j = wid % 4
    pltpu.sync_copy(idx_hbm.at[b], idx_v)

    def do_row(src_row, dst_row, prev):
        pltpu.sync_copy(src_row, row_v)
        if prev is not None:
            prev[0].wait()
            prev[1].wait()
        _gather_half(row_v, idx_v, oa_v, 0)
        da = pltpu.async_copy(oa_v, dst_row.at[pl.ds(0, HALF)], sem_a)
        _gather_half(row_v, idx_v, ob_v, 1)
        db = pltpu.async_copy(ob_v, dst_row.at[pl.ds(HALF, HALF)], sem_b)
        return (da, db)

    prev = None
    for ci in range(C // 4):
        c = j * (C // 4) + ci
        prev = do_row(y_hbm.at[b, c], out_y.at[b, c], prev)
    prev[0].wait()
    prev[1].wait()

    @pl.when(j < CX)
    def _():
        pltpu.sync_copy(x_hbm.at[b, j], row_v)
        _gather_half(row_v, idx_v, oa_v, 0)
        _gather_half(row_v, idx_v, ob_v, 1)
        pltpu.sync_copy(oa_v, out_x.at[b, j, pl.ds(0, HALF)])
        pltpu.sync_copy(ob_v, out_x.at[b, j, pl.ds(HALF, HALF)])


@functools.cache
def _make_sc_gather():
    return pl.kernel(
        _sc_gather_body,
        out_type=(
            jax.ShapeDtypeStruct((B, CX, K_TOP), jnp.float32),
            jax.ShapeDtypeStruct((B, C, K_TOP), jnp.float32),
        ),
        mesh=plsc.VectorSubcoreMesh(core_axis_name="c", subcore_axis_name="s"),
        compiler_params=pltpu.CompilerParams(needs_layout_passes=False),
        scratch_types=[
            pltpu.VMEM((N,), jnp.float32),
            pltpu.VMEM((K_TOP,), jnp.int32),
            pltpu.VMEM((HALF,), jnp.float32),
            pltpu.VMEM((HALF,), jnp.float32),
            pltpu.SemaphoreType.DMA,
            pltpu.SemaphoreType.DMA,
        ],
    )


def kernel(x, y):
    keys = _maxkey(y)
    keys = keys.reshape(B, ROWS, LANES)
    idx = jnp.swapaxes(_sort(keys), 1, 2).reshape(B, K_TOP)
    top_k_xyz, top_k_points = _make_sc_gather()(x, y, idx)
    return (top_k_xyz, top_k_points)


# speedup vs baseline: 1.1553x; 1.0048x over previous
import functools

import jax
import jax.numpy as jnp
from jax import lax
from jax.experimental import pallas as pl
from jax.experimental.pallas import tpu as pltpu
from jax.experimental.pallas import tpu_sc as plsc

B, C, N = 8, 64, 65536
K_TOP = 16384
CHUNK = 8192
NCHUNK = N // CHUNK
ROWS, LANES = 512, 128


def _maxkey_body(y_ref, out_ref):
    yv = y_ref[0]
    z = jnp.max(yv, axis=0, keepdims=True)
    bits = jax.lax.bitcast_convert_type(z, jnp.int32)
    key = jnp.where(bits >= 0, ~bits, bits ^ jnp.int32(-2147483648))
    out_ref[0] = key


def _maxkey(y):
    return pl.pallas_call(
        _maxkey_body,
        grid=(B, NCHUNK),
        in_specs=[pl.BlockSpec((1, C, CHUNK), lambda b, c: (b, 0, c))],
        out_specs=pl.BlockSpec((1, 1, CHUNK), lambda b, c: (b * NCHUNK + c, 0, 0)),
        out_shape=jax.ShapeDtypeStruct((B * NCHUNK, 1, CHUNK), jnp.int32),
    )(y)


def _rot(a, sh, axis):
    n = a.shape[axis]
    sh = sh % n
    if sh == 0:
        return a
    idx_hi = [slice(None)] * a.ndim
    idx_lo = [slice(None)] * a.ndim
    idx_hi[axis] = slice(sh, None)
    idx_lo[axis] = slice(None, sh)
    return jnp.concatenate([a[tuple(idx_hi)], a[tuple(idx_lo)]], axis=axis)


CHROWS = 128
_TR_MIN_M = 512


def _lex_bLTa(Ka, Ia, Kb, Ib):
    return (Kb < Ka) | ((Kb == Ka) & (Ib < Ia))


def _pair_rows(arr, s):
    sh = arr.shape
    g = sh[1] // (2 * s)
    v = arr.reshape(sh[0], g, 2, s, sh[2])
    return v[:, :, 0], v[:, :, 1]


def _unpair_rows(a, b):
    sh = a.shape
    v = jnp.concatenate([a[:, :, None], b[:, :, None]], axis=2)
    return v.reshape(sh[0], sh[1] * 2 * sh[2], sh[3])


def _sublane_stage(K, I, dirA, s):
    Ka, Kb = _pair_rows(K, s)
    Ia, Ib = _pair_rows(I, s)
    da, _ = _pair_rows(jnp.broadcast_to(dirA, K.shape), s)
    swap = _lex_bLTa(Ka, Ia, Kb, Ib) == da
    nKa = jnp.where(swap, Kb, Ka)
    nKb = jnp.where(swap, Ka, Kb)
    nIa = jnp.where(swap, Ib, Ia)
    nIb = jnp.where(swap, Ia, Ib)
    return _unpair_rows(nKa, nKb), _unpair_rows(nIa, nIb)


def _rot(a, sh, axis):
    n = a.shape[axis]
    return pltpu.roll(a, (-sh) % n, axis)


def _roll_stage(K, I, dirA, st, axis):
    iot = lax.broadcasted_iota(jnp.int32, K.shape[1:], axis - 1)[None]
    low = (iot & st) == 0
    Kp = jnp.where(low, _rot(K, st, axis), _rot(K, -st, axis))
    Ip = jnp.where(low, _rot(I, st, axis), _rot(I, -st, axis))
    pLTm = _lex_bLTa(K, I, Kp, Ip)
    takeP = (low == dirA) == pLTm
    return jnp.where(takeP, Kp, K), jnp.where(takeP, Ip, I)


def _stage(K, I, dirA, s):
    if 8 <= s <= 64:
        return _sublane_stage(K, I, dirA, s)
    if s <= 4:
        return _roll_stage(K, I, dirA, s, 1)
    return _roll_stage(K, I, dirA, s // CHROWS, 2)


def _tr(a):
    sh = a.shape
    v = a.reshape(sh[0], sh[1] // CHROWS, CHROWS, LANES)
    return jnp.swapaxes(v, 2, 3).reshape(sh)


def _merge_tail(K, I, dirA, dirT, m):
    KT, IT = _tr(K), _tr(I)
    sl = m // (2 * CHROWS)
    while sl >= 1:
        KT, IT = _stage(KT, IT, dirT, sl)
        sl //= 2
    K, I = _tr(KT), _tr(IT)
    s = 64
    while s >= 1:
        K, I = _stage(K, I, dirA, s)
        s //= 2
    return K, I


def _prune_pairs(K, I):
    sh = K.shape
    g = sh[1] // (2 * CHROWS)
    Kv = K.reshape(sh[0], g, 2, CHROWS, sh[2])
    Iv = I.reshape(sh[0], g, 2, CHROWS, sh[2])
    Ka, Kb = Kv[:, :, 0], Kv[:, :, 1]
    Ia, Ib = Iv[:, :, 0], Iv[:, :, 1]
    t = _lex_bLTa(Ka, Ia, Kb, Ib)
    nK = jnp.where(t, Kb, Ka).reshape(sh[0], g * CHROWS, sh[2])
    nI = jnp.where(t, Ib, Ia).reshape(sh[0], g * CHROWS, sh[2])
    return nK, nI


def _sort_body(k_ref, out_ref):
    K = k_ref[...]
    Pr = lax.broadcasted_iota(jnp.int32, (1, ROWS, LANES), 1)
    Pc = lax.broadcasted_iota(jnp.int32, (1, ROWS, LANES), 2)
    I = jnp.broadcast_to(Pr * LANES + Pc, K.shape)
    J = (Pr >> 7) * K_TOP + Pc * CHROWS + (Pr & 127)
    JT = (Pr >> 7) * K_TOP + (Pr & 127) * CHROWS + Pc

    m = 2
    while m < _TR_MIN_M:
        dirA = (J & m) == 0
        s = m // 2
        while s >= 1:
            K, I = _stage(K, I, dirA, s)
            s //= 2
        m *= 2
    while m <= K_TOP:
        K, I = _merge_tail(K, I, (J & m) == 0, (JT & m) == 0, m)
        m *= 2

    K, I = _prune_pairs(K, I)
    dirA = lax.broadcasted_iota(jnp.int32, (1, 2 * CHROWS, 1), 1) < CHROWS
    K, I = _merge_tail(K, I, dirA, dirA, K_TOP)
    K, I = _prune_pairs(K, I)
    dirA = jnp.full((1, 1, 1), True)
    K, I = _merge_tail(K, I, dirA, dirA, K_TOP)
    out_ref[...] = I


def _sort(keys):
    return pl.pallas_call(
        _sort_body,
        in_specs=[pl.BlockSpec((B, ROWS, LANES), lambda: (0, 0, 0))],
        out_specs=pl.BlockSpec((B, CHROWS, LANES), lambda: (0, 0, 0)),
        out_shape=jax.ShapeDtypeStruct((B, CHROWS, LANES), jnp.int32),
    )(keys)


CX = 3


HALF = K_TOP // 2


def _gather_half(row_v, idx_v, out_v, h):
    @plsc.parallel_loop(h * HALF, (h + 1) * HALF, 16, unroll=16)
    def _(i):
        iv = idx_v[pl.ds(i, 16)]
        out_v[pl.ds(i - h * HALF, 16)] = plsc.load_gather(row_v, [iv])


def _sc_gather_body(
    x_hbm, y_hbm, idx_hbm, out_x, out_y, row_v, idx_v, oa_v, ob_v, sem_a, sem_b
):
    cid = lax.axis_index("c")
    sid = lax.axis_index("s")
    wid = sid * 2 + cid
    b = wid // 4
    j = wid % 4
    pltpu.sync_copy(idx_hbm.at[b], idx_v)

    def do_row(src_row, dst_row, prev):
        pltpu.sync_copy(src_row, row_v)
        if prev is not None:
            prev[0].wait()
            prev[1].wait()
        _gather_half(row_v, idx_v, oa_v, 0)
        da = pltpu.async_copy(oa_v, dst_row.at[pl.ds(0, HALF)], sem_a)
        _gather_half(row_v, idx_v, ob_v, 1)
        db = pltpu.async_copy(ob_v, dst_row.at[pl.ds(HALF, HALF)], sem_b)
        return (da, db)

    prev = None
    for ci in range(C // 4):
        c = j * (C // 4) + ci
        prev = do_row(y_hbm.at[b, c], out_y.at[b, c], prev)
    prev[0].wait()
    prev[1].wait()

    @pl.when(j < CX)
    def _():
        pltpu.sync_copy(x_hbm.at[b, j], row_v)
        _gather_half(row_v, idx_v, oa_v, 0)
        _gather_half(row_v, idx_v, ob_v, 1)
        pltpu.sync_copy(oa_v, out_x.at[b, j, pl.ds(0, HALF)])
        pltpu.sync_copy(ob_v, out_x.at[b, j, pl.ds(HALF, HALF)])


@functools.cache
def _make_sc_gather():
    return pl.kernel(
        _sc_gather_body,
        out_type=(
            jax.ShapeDtypeStruct((B, CX, K_TOP), jnp.float32),
            jax.ShapeDtypeStruct((B, C, K_TOP), jnp.float32),
        ),
        mesh=plsc.VectorSubcoreMesh(core_axis_name="c", subcore_axis_name="s"),
        compiler_params=pltpu.CompilerParams(needs_layout_passes=False),
        scratch_types=[
            pltpu.VMEM((N,), jnp.float32),
            pltpu.VMEM((K_TOP,), jnp.int32),
            pltpu.VMEM((HALF,), jnp.float32),
            pltpu.VMEM((HALF,), jnp.float32),
            pltpu.SemaphoreType.DMA,
            pltpu.SemaphoreType.DMA,
        ],
    )


def kernel(x, y):
    keys = _maxkey(y)
    keys = keys.reshape(B, ROWS, LANES)
    idx = jnp.swapaxes(_sort(keys), 1, 2).reshape(B, K_TOP)
    top_k_xyz, top_k_points = _make_sc_gather()(x, y, idx)
    return (top_k_xyz, top_k_points)


# speedup vs baseline: 1.1747x; 1.0168x over previous
import functools

import jax
import jax.numpy as jnp
from jax import lax
from jax.experimental import pallas as pl
from jax.experimental.pallas import tpu as pltpu
from jax.experimental.pallas import tpu_sc as plsc

B, C, N = 8, 64, 65536
K_TOP = 16384
CHUNK = 8192
NCHUNK = N // CHUNK
ROWS, LANES = 512, 128


def _maxkey_body(y_ref, out_ref):
    yv = y_ref[0]
    z = jnp.max(yv, axis=0, keepdims=True)
    bits = jax.lax.bitcast_convert_type(z, jnp.int32)
    key = jnp.where(bits >= 0, ~bits, bits ^ jnp.int32(-2147483648))
    out_ref[0] = key


def _maxkey(y):
    return pl.pallas_call(
        _maxkey_body,
        grid=(B, NCHUNK),
        in_specs=[pl.BlockSpec((1, C, CHUNK), lambda b, c: (b, 0, c))],
        out_specs=pl.BlockSpec((1, 1, CHUNK), lambda b, c: (b * NCHUNK + c, 0, 0)),
        out_shape=jax.ShapeDtypeStruct((B * NCHUNK, 1, CHUNK), jnp.int32),
    )(y)


def _rot(a, sh, axis):
    n = a.shape[axis]
    sh = sh % n
    if sh == 0:
        return a
    idx_hi = [slice(None)] * a.ndim
    idx_lo = [slice(None)] * a.ndim
    idx_hi[axis] = slice(sh, None)
    idx_lo[axis] = slice(None, sh)
    return jnp.concatenate([a[tuple(idx_hi)], a[tuple(idx_lo)]], axis=axis)


CHROWS = 128
_TR_MIN_M = 512


def _lex_bLTa(Ka, Ia, Kb, Ib):
    return (Kb < Ka) | ((Kb == Ka) & (Ib < Ia))


def _pair_rows(arr, s):
    sh = arr.shape
    g = sh[1] // (2 * s)
    v = arr.reshape(sh[0], g, 2, s, sh[2])
    return v[:, :, 0], v[:, :, 1]


def _unpair_rows(a, b):
    sh = a.shape
    v = jnp.concatenate([a[:, :, None], b[:, :, None]], axis=2)
    return v.reshape(sh[0], sh[1] * 2 * sh[2], sh[3])


def _sublane_stage(K, I, dirA, s):
    Ka, Kb = _pair_rows(K, s)
    Ia, Ib = _pair_rows(I, s)
    da, _ = _pair_rows(jnp.broadcast_to(dirA, K.shape), s)
    swap = _lex_bLTa(Ka, Ia, Kb, Ib) == da
    nKa = jnp.where(swap, Kb, Ka)
    nKb = jnp.where(swap, Ka, Kb)
    nIa = jnp.where(swap, Ib, Ia)
    nIb = jnp.where(swap, Ia, Ib)
    return _unpair_rows(nKa, nKb), _unpair_rows(nIa, nIb)


def _rot(a, sh, axis):
    n = a.shape[axis]
    return pltpu.roll(a, (-sh) % n, axis)


def _roll_stage(K, I, dirA, st, axis):
    iot = lax.broadcasted_iota(jnp.int32, K.shape[1:], axis - 1)[None]
    low = (iot & st) == 0
    Kp = jnp.where(low, _rot(K, st, axis), _rot(K, -st, axis))
    Ip = jnp.where(low, _rot(I, st, axis), _rot(I, -st, axis))
    pLTm = _lex_bLTa(K, I, Kp, Ip)
    takeP = (low == dirA) == pLTm
    return jnp.where(takeP, Kp, K), jnp.where(takeP, Ip, I)


def _stage(K, I, dirA, s):
    if 8 <= s <= 64:
        return _sublane_stage(K, I, dirA, s)
    if s <= 4:
        return _roll_stage(K, I, dirA, s, 1)
    return _roll_stage(K, I, dirA, s // CHROWS, 2)


def _tr(a):
    sh = a.shape
    v = a.reshape(sh[0], sh[1] // CHROWS, CHROWS, LANES)
    return jnp.swapaxes(v, 2, 3).reshape(sh)


def _merge_tail(K, I, dirA, dirT, m):
    KT, IT = _tr(K), _tr(I)
    sl = m // (2 * CHROWS)
    while sl >= 1:
        KT, IT = _stage(KT, IT, dirT, sl)
        sl //= 2
    K, I = _tr(KT), _tr(IT)
    s = 64
    while s >= 1:
        K, I = _stage(K, I, dirA, s)
        s //= 2
    return K, I


def _prune_pairs(K, I):
    sh = K.shape
    g = sh[1] // (2 * CHROWS)
    Kv = K.reshape(sh[0], g, 2, CHROWS, sh[2])
    Iv = I.reshape(sh[0], g, 2, CHROWS, sh[2])
    Ka, Kb = Kv[:, :, 0], Kv[:, :, 1]
    Ia, Ib = Iv[:, :, 0], Iv[:, :, 1]
    t = _lex_bLTa(Ka, Ia, Kb, Ib)
    nK = jnp.where(t, Kb, Ka).reshape(sh[0], g * CHROWS, sh[2])
    nI = jnp.where(t, Ib, Ia).reshape(sh[0], g * CHROWS, sh[2])
    return nK, nI


def _sort_body(k_ref, out_ref):
    K = k_ref[...]
    Pr = lax.broadcasted_iota(jnp.int32, (1, ROWS, LANES), 1)
    Pc = lax.broadcasted_iota(jnp.int32, (1, ROWS, LANES), 2)
    I = jnp.broadcast_to(Pr * LANES + Pc, K.shape)
    J = (Pr >> 7) * K_TOP + Pc * CHROWS + (Pr & 127)
    JT = (Pr >> 7) * K_TOP + (Pr & 127) * CHROWS + Pc

    m = 2
    while m < _TR_MIN_M:
        dirA = (J & m) == 0
        s = m // 2
        while s >= 1:
            K, I = _stage(K, I, dirA, s)
            s //= 2
        m *= 2
    while m <= K_TOP:
        K, I = _merge_tail(K, I, (J & m) == 0, (JT & m) == 0, m)
        m *= 2

    K, I = _prune_pairs(K, I)
    dirA = lax.broadcasted_iota(jnp.int32, (1, 2 * CHROWS, 1), 1) < CHROWS
    K, I = _merge_tail(K, I, dirA, dirA, K_TOP)
    K, I = _prune_pairs(K, I)
    dirA = jnp.full((1, 1, 1), True)
    K, I = _merge_tail(K, I, dirA, dirA, K_TOP)
    out_ref[...] = I


def _sort(keys):
    nb = keys.shape[0]
    return pl.pallas_call(
        _sort_body,
        in_specs=[pl.BlockSpec((nb, ROWS, LANES), lambda: (0, 0, 0))],
        out_specs=pl.BlockSpec((nb, CHROWS, LANES), lambda: (0, 0, 0)),
        out_shape=jax.ShapeDtypeStruct((nb, CHROWS, LANES), jnp.int32),
    )(keys)


CX = 3


HALF = K_TOP // 2


def _gather_half(row_v, idx_v, out_v, h):
    @plsc.parallel_loop(h * HALF, (h + 1) * HALF, 16, unroll=16)
    def _(i):
        iv = idx_v[pl.ds(i, 16)]
        out_v[pl.ds(i - h * HALF, 16)] = plsc.load_gather(row_v, [iv])


def _sc_gather_body(
    b0, x_hbm, y_hbm, idx_hbm, out_x, out_y, row_v, idx_v, oa_v, ob_v, sem_a, sem_b
):
    cid = lax.axis_index("c")
    sid = lax.axis_index("s")
    wid = sid * 2 + cid
    nw = 8
    b = wid // nw
    j = wid % nw
    pltpu.sync_copy(idx_hbm.at[b], idx_v)

    def do_row(src_row, dst_row, prev):
        pltpu.sync_copy(src_row, row_v)
        if prev is not None:
            prev[0].wait()
            prev[1].wait()
        _gather_half(row_v, idx_v, oa_v, 0)
        da = pltpu.async_copy(oa_v, dst_row.at[pl.ds(0, HALF)], sem_a)
        _gather_half(row_v, idx_v, ob_v, 1)
        db = pltpu.async_copy(ob_v, dst_row.at[pl.ds(HALF, HALF)], sem_b)
        return (da, db)

    prev = None
    for ci in range(C // nw):
        c = j * (C // nw) + ci
        prev = do_row(y_hbm.at[b0 + b, c], out_y.at[b, c], prev)
    prev[0].wait()
    prev[1].wait()

    @pl.when(j < CX)
    def _():
        pltpu.sync_copy(x_hbm.at[b0 + b, j], row_v)
        _gather_half(row_v, idx_v, oa_v, 0)
        _gather_half(row_v, idx_v, ob_v, 1)
        pltpu.sync_copy(oa_v, out_x.at[b, j, pl.ds(0, HALF)])
        pltpu.sync_copy(ob_v, out_x.at[b, j, pl.ds(HALF, HALF)])


@functools.cache
def _make_sc_gather(b0):
    return pl.kernel(
        functools.partial(_sc_gather_body, b0),
        out_type=(
            jax.ShapeDtypeStruct((B // 2, CX, K_TOP), jnp.float32),
            jax.ShapeDtypeStruct((B // 2, C, K_TOP), jnp.float32),
        ),
        mesh=plsc.VectorSubcoreMesh(core_axis_name="c", subcore_axis_name="s"),
        compiler_params=pltpu.CompilerParams(needs_layout_passes=False),
        scratch_types=[
            pltpu.VMEM((N,), jnp.float32),
            pltpu.VMEM((K_TOP,), jnp.int32),
            pltpu.VMEM((HALF,), jnp.float32),
            pltpu.VMEM((HALF,), jnp.float32),
            pltpu.SemaphoreType.DMA,
            pltpu.SemaphoreType.DMA,
        ],
    )


def kernel(x, y):
    keys = _maxkey(y)
    keys = keys.reshape(B, ROWS, LANES)
    h = B // 2
    idx_a = jnp.swapaxes(_sort(keys[:h]), 1, 2).reshape(h, K_TOP)
    gx_a, gy_a = _make_sc_gather(0)(x, y, idx_a)
    idx_b = jnp.swapaxes(_sort(keys[h:]), 1, 2).reshape(h, K_TOP)
    gx_b, gy_b = _make_sc_gather(h)(x, y, idx_b)
    top_k_xyz = jnp.concatenate([gx_a, gx_b], axis=0)
    top_k_points = jnp.concatenate([gy_a, gy_b], axis=0)
    return (top_k_xyz, top_k_points)


# speedup vs baseline: 1.1757x; 1.0009x over previous
import functools

import jax
import jax.numpy as jnp
from jax import lax
from jax.experimental import pallas as pl
from jax.experimental.pallas import tpu as pltpu
from jax.experimental.pallas import tpu_sc as plsc

B, C, N = 8, 64, 65536
K_TOP = 16384
CHUNK = 8192
NCHUNK = N // CHUNK
ROWS, LANES = 512, 128


def _maxkey_body(y_ref, out_ref):
    yv = y_ref[0]
    z = jnp.max(yv, axis=0, keepdims=True)
    bits = jax.lax.bitcast_convert_type(z, jnp.int32)
    key = jnp.where(bits >= 0, ~bits, bits ^ jnp.int32(-2147483648))
    out_ref[0] = key


def _maxkey(y):
    return pl.pallas_call(
        _maxkey_body,
        grid=(B, NCHUNK),
        in_specs=[pl.BlockSpec((1, C, CHUNK), lambda b, c: (b, 0, c))],
        out_specs=pl.BlockSpec((1, 1, CHUNK), lambda b, c: (b * NCHUNK + c, 0, 0)),
        out_shape=jax.ShapeDtypeStruct((B * NCHUNK, 1, CHUNK), jnp.int32),
    )(y)


def _rot(a, sh, axis):
    n = a.shape[axis]
    sh = sh % n
    if sh == 0:
        return a
    idx_hi = [slice(None)] * a.ndim
    idx_lo = [slice(None)] * a.ndim
    idx_hi[axis] = slice(sh, None)
    idx_lo[axis] = slice(None, sh)
    return jnp.concatenate([a[tuple(idx_hi)], a[tuple(idx_lo)]], axis=axis)


CHROWS = 128
_TR_MIN_M = 512


def _lex_bLTa(Ka, Ia, Kb, Ib):
    return (Kb < Ka) | ((Kb == Ka) & (Ib < Ia))


def _pair_rows(arr, s):
    sh = arr.shape
    g = sh[1] // (2 * s)
    v = arr.reshape(sh[0], g, 2, s, sh[2])
    return v[:, :, 0], v[:, :, 1]


def _unpair_rows(a, b):
    sh = a.shape
    v = jnp.concatenate([a[:, :, None], b[:, :, None]], axis=2)
    return v.reshape(sh[0], sh[1] * 2 * sh[2], sh[3])


def _sublane_stage(K, I, dirA, s):
    Ka, Kb = _pair_rows(K, s)
    Ia, Ib = _pair_rows(I, s)
    da, _ = _pair_rows(jnp.broadcast_to(dirA, K.shape), s)
    swap = _lex_bLTa(Ka, Ia, Kb, Ib) == da
    nKa = jnp.where(swap, Kb, Ka)
    nKb = jnp.where(swap, Ka, Kb)
    nIa = jnp.where(swap, Ib, Ia)
    nIb = jnp.where(swap, Ia, Ib)
    return _unpair_rows(nKa, nKb), _unpair_rows(nIa, nIb)


def _rot(a, sh, axis):
    n = a.shape[axis]
    return pltpu.roll(a, (-sh) % n, axis)


def _roll_stage(K, I, dirA, st, axis):
    iot = lax.broadcasted_iota(jnp.int32, K.shape[1:], axis - 1)[None]
    low = (iot & st) == 0
    Kp = jnp.where(low, _rot(K, st, axis), _rot(K, -st, axis))
    Ip = jnp.where(low, _rot(I, st, axis), _rot(I, -st, axis))
    pLTm = _lex_bLTa(K, I, Kp, Ip)
    takeP = (low == dirA) == pLTm
    return jnp.where(takeP, Kp, K), jnp.where(takeP, Ip, I)


def _stage(K, I, dirA, s):
    if 8 <= s <= 64:
        return _sublane_stage(K, I, dirA, s)
    if s <= 4:
        return _roll_stage(K, I, dirA, s, 1)
    return _roll_stage(K, I, dirA, s // CHROWS, 2)


def _tr(a):
    sh = a.shape
    v = a.reshape(sh[0], sh[1] // CHROWS, CHROWS, LANES)
    return jnp.swapaxes(v, 2, 3).reshape(sh)


def _merge_tail(K, I, dirA, dirT, m):
    KT, IT = _tr(K), _tr(I)
    sl = m // (2 * CHROWS)
    while sl >= 1:
        KT, IT = _stage(KT, IT, dirT, sl)
        sl //= 2
    K, I = _tr(KT), _tr(IT)
    s = 64
    while s >= 1:
        K, I = _stage(K, I, dirA, s)
        s //= 2
    return K, I


def _prune_pairs(K, I):
    sh = K.shape
    g = sh[1] // (2 * CHROWS)
    Kv = K.reshape(sh[0], g, 2, CHROWS, sh[2])
    Iv = I.reshape(sh[0], g, 2, CHROWS, sh[2])
    Ka, Kb = Kv[:, :, 0], Kv[:, :, 1]
    Ia, Ib = Iv[:, :, 0], Iv[:, :, 1]
    t = _lex_bLTa(Ka, Ia, Kb, Ib)
    nK = jnp.where(t, Kb, Ka).reshape(sh[0], g * CHROWS, sh[2])
    nI = jnp.where(t, Ib, Ia).reshape(sh[0], g * CHROWS, sh[2])
    return nK, nI


def _sort_body(k_ref, out_ref):
    K = k_ref[...]
    Pr = lax.broadcasted_iota(jnp.int32, (1, ROWS, LANES), 1)
    Pc = lax.broadcasted_iota(jnp.int32, (1, ROWS, LANES), 2)
    I = jnp.broadcast_to(Pr * LANES + Pc, K.shape)
    J = (Pr >> 7) * K_TOP + Pc * CHROWS + (Pr & 127)
    JT = (Pr >> 7) * K_TOP + (Pr & 127) * CHROWS + Pc

    m = 2
    while m < _TR_MIN_M:
        dirA = (J & m) == 0
        s = m // 2
        while s >= 1:
            K, I = _stage(K, I, dirA, s)
            s //= 2
        m *= 2
    while m <= K_TOP:
        K, I = _merge_tail(K, I, (J & m) == 0, (JT & m) == 0, m)
        m *= 2

    K, I = _prune_pairs(K, I)
    dirA = lax.broadcasted_iota(jnp.int32, (1, 2 * CHROWS, 1), 1) < CHROWS
    K, I = _merge_tail(K, I, dirA, dirA, K_TOP)
    K, I = _prune_pairs(K, I)
    dirA = jnp.full((1, 1, 1), True)
    K, I = _merge_tail(K, I, dirA, dirA, K_TOP)
    out_ref[...] = I


def _sort(keys):
    nb = keys.shape[0]
    return pl.pallas_call(
        _sort_body,
        in_specs=[pl.BlockSpec((nb, ROWS, LANES), lambda: (0, 0, 0))],
        out_specs=pl.BlockSpec((nb, CHROWS, LANES), lambda: (0, 0, 0)),
        out_shape=jax.ShapeDtypeStruct((nb, CHROWS, LANES), jnp.int32),
        cost_estimate=pl.CostEstimate(
            flops=2_000_000_000, transcendentals=0, bytes_accessed=16 << 20
        ),
    )(keys)


CX = 3


HALF = K_TOP // 2


def _gather_half(row_v, idx_v, out_v, h):
    @plsc.parallel_loop(h * HALF, (h + 1) * HALF, 16, unroll=16)
    def _(i):
        iv = idx_v[pl.ds(i, 16)]
        out_v[pl.ds(i - h * HALF, 16)] = plsc.load_gather(row_v, [iv])


def _sc_gather_body(
    b0, x_hbm, y_hbm, idx_hbm, out_x, out_y, row_v, idx_v, oa_v, ob_v, sem_a, sem_b
):
    cid = lax.axis_index("c")
    sid = lax.axis_index("s")
    wid = sid * 2 + cid
    nw = 8
    b = wid // nw
    j = wid % nw
    pltpu.sync_copy(idx_hbm.at[b], idx_v)

    def do_row(src_row, dst_row, prev):
        pltpu.sync_copy(src_row, row_v)
        if prev is not None:
            prev[0].wait()
            prev[1].wait()
        _gather_half(row_v, idx_v, oa_v, 0)
        da = pltpu.async_copy(oa_v, dst_row.at[pl.ds(0, HALF)], sem_a)
        _gather_half(row_v, idx_v, ob_v, 1)
        db = pltpu.async_copy(ob_v, dst_row.at[pl.ds(HALF, HALF)], sem_b)
        return (da, db)

    prev = None
    for ci in range(C // nw):
        c = j * (C // nw) + ci
        prev = do_row(y_hbm.at[b0 + b, c], out_y.at[b, c], prev)
    prev[0].wait()
    prev[1].wait()

    @pl.when(j < CX)
    def _():
        pltpu.sync_copy(x_hbm.at[b0 + b, j], row_v)
        _gather_half(row_v, idx_v, oa_v, 0)
        _gather_half(row_v, idx_v, ob_v, 1)
        pltpu.sync_copy(oa_v, out_x.at[b, j, pl.ds(0, HALF)])
        pltpu.sync_copy(ob_v, out_x.at[b, j, pl.ds(HALF, HALF)])


@functools.cache
def _make_sc_gather(b0):
    return pl.kernel(
        functools.partial(_sc_gather_body, b0),
        out_type=(
            jax.ShapeDtypeStruct((B // 2, CX, K_TOP), jnp.float32),
            jax.ShapeDtypeStruct((B // 2, C, K_TOP), jnp.float32),
        ),
        mesh=plsc.VectorSubcoreMesh(core_axis_name="c", subcore_axis_name="s"),
        compiler_params=pltpu.CompilerParams(needs_layout_passes=False),
        scratch_types=[
            pltpu.VMEM((N,), jnp.float32),
            pltpu.VMEM((K_TOP,), jnp.int32),
            pltpu.VMEM((HALF,), jnp.float32),
            pltpu.VMEM((HALF,), jnp.float32),
            pltpu.SemaphoreType.DMA,
            pltpu.SemaphoreType.DMA,
        ],
    )


def kernel(x, y):
    keys = _maxkey(y)
    keys = keys.reshape(B, ROWS, LANES)
    h = B // 2
    idx_a = jnp.swapaxes(_sort(keys[:h]), 1, 2).reshape(h, K_TOP)
    gx_a, gy_a = _make_sc_gather(0)(x, y, idx_a)
    idx_b = jnp.swapaxes(_sort(keys[h:]), 1, 2).reshape(h, K_TOP)
    gx_b, gy_b = _make_sc_gather(h)(x, y, idx_b)
    top_k_xyz = jnp.concatenate([gx_a, gx_b], axis=0)
    top_k_points = jnp.concatenate([gy_a, gy_b], axis=0)
    return (top_k_xyz, top_k_points)


# speedup vs baseline: 1.1810x; 1.0045x over previous
import functools

import jax
import jax.numpy as jnp
from jax import lax
from jax.experimental import pallas as pl
from jax.experimental.pallas import tpu as pltpu
from jax.experimental.pallas import tpu_sc as plsc

B, C, N = 8, 64, 65536
K_TOP = 16384
CHUNK = 8192
NCHUNK = N // CHUNK
ROWS, LANES = 512, 128


def _maxkey_body(y_ref, out_ref):
    yv = y_ref[0]
    z = jnp.max(yv, axis=0, keepdims=True)
    bits = jax.lax.bitcast_convert_type(z, jnp.int32)
    key = jnp.where(bits >= 0, ~bits, bits ^ jnp.int32(-2147483648))
    out_ref[0] = key


def _maxkey(y):
    return pl.pallas_call(
        _maxkey_body,
        grid=(B, NCHUNK),
        in_specs=[pl.BlockSpec((1, C, CHUNK), lambda b, c: (b, 0, c))],
        out_specs=pl.BlockSpec((1, 1, CHUNK), lambda b, c: (b * NCHUNK + c, 0, 0)),
        out_shape=jax.ShapeDtypeStruct((B * NCHUNK, 1, CHUNK), jnp.int32),
    )(y)


def _rot(a, sh, axis):
    n = a.shape[axis]
    sh = sh % n
    if sh == 0:
        return a
    idx_hi = [slice(None)] * a.ndim
    idx_lo = [slice(None)] * a.ndim
    idx_hi[axis] = slice(sh, None)
    idx_lo[axis] = slice(None, sh)
    return jnp.concatenate([a[tuple(idx_hi)], a[tuple(idx_lo)]], axis=axis)


CHROWS = 128
_TR_MIN_M = 256


def _lex_bLTa(Ka, Ia, Kb, Ib):
    return (Kb < Ka) | ((Kb == Ka) & (Ib < Ia))


def _pair_rows(arr, s):
    sh = arr.shape
    g = sh[1] // (2 * s)
    v = arr.reshape(sh[0], g, 2, s, sh[2])
    return v[:, :, 0], v[:, :, 1]


def _unpair_rows(a, b):
    sh = a.shape
    v = jnp.concatenate([a[:, :, None], b[:, :, None]], axis=2)
    return v.reshape(sh[0], sh[1] * 2 * sh[2], sh[3])


def _sublane_stage(K, I, dirA, s):
    Ka, Kb = _pair_rows(K, s)
    Ia, Ib = _pair_rows(I, s)
    da, _ = _pair_rows(jnp.broadcast_to(dirA, K.shape), s)
    swap = _lex_bLTa(Ka, Ia, Kb, Ib) == da
    nKa = jnp.where(swap, Kb, Ka)
    nKb = jnp.where(swap, Ka, Kb)
    nIa = jnp.where(swap, Ib, Ia)
    nIb = jnp.where(swap, Ia, Ib)
    return _unpair_rows(nKa, nKb), _unpair_rows(nIa, nIb)


def _rot(a, sh, axis):
    n = a.shape[axis]
    return pltpu.roll(a, (-sh) % n, axis)


def _roll_stage(K, I, dirA, st, axis):
    iot = lax.broadcasted_iota(jnp.int32, K.shape[1:], axis - 1)[None]
    low = (iot & st) == 0
    Kp = jnp.where(low, _rot(K, st, axis), _rot(K, -st, axis))
    Ip = jnp.where(low, _rot(I, st, axis), _rot(I, -st, axis))
    pLTm = _lex_bLTa(K, I, Kp, Ip)
    takeP = (low == dirA) == pLTm
    return jnp.where(takeP, Kp, K), jnp.where(takeP, Ip, I)


def _stage(K, I, dirA, s):
    if 8 <= s <= 64:
        return _sublane_stage(K, I, dirA, s)
    if s <= 4:
        return _roll_stage(K, I, dirA, s, 1)
    return _roll_stage(K, I, dirA, s // CHROWS, 2)


def _tr(a):
    sh = a.shape
    v = a.reshape(sh[0], sh[1] // CHROWS, CHROWS, LANES)
    return jnp.swapaxes(v, 2, 3).reshape(sh)


def _merge_tail(K, I, dirA, dirT, m):
    KT, IT = _tr(K), _tr(I)
    sl = m // (2 * CHROWS)
    while sl >= 1:
        KT, IT = _stage(KT, IT, dirT, sl)
        sl //= 2
    K, I = _tr(KT), _tr(IT)
    s = 64
    while s >= 1:
        K, I = _stage(K, I, dirA, s)
        s //= 2
    return K, I


def _prune_pairs(K, I):
    sh = K.shape
    g = sh[1] // (2 * CHROWS)
    Kv = K.reshape(sh[0], g, 2, CHROWS, sh[2])
    Iv = I.reshape(sh[0], g, 2, CHROWS, sh[2])
    Ka, Kb = Kv[:, :, 0], Kv[:, :, 1]
    Ia, Ib = Iv[:, :, 0], Iv[:, :, 1]
    t = _lex_bLTa(Ka, Ia, Kb, Ib)
    nK = jnp.where(t, Kb, Ka).reshape(sh[0], g * CHROWS, sh[2])
    nI = jnp.where(t, Ib, Ia).reshape(sh[0], g * CHROWS, sh[2])
    return nK, nI


def _sort_body(k_ref, out_ref):
    K = k_ref[...]
    Pr = lax.broadcasted_iota(jnp.int32, (1, ROWS, LANES), 1)
    Pc = lax.broadcasted_iota(jnp.int32, (1, ROWS, LANES), 2)
    I = jnp.broadcast_to(Pr * LANES + Pc, K.shape)
    J = (Pr >> 7) * K_TOP + Pc * CHROWS + (Pr & 127)
    JT = (Pr >> 7) * K_TOP + (Pr & 127) * CHROWS + Pc

    m = 2
    while m < _TR_MIN_M:
        dirA = (J & m) == 0
        s = m // 2
        while s >= 1:
            K, I = _stage(K, I, dirA, s)
            s //= 2
        m *= 2
    while m <= K_TOP:
        K, I = _merge_tail(K, I, (J & m) == 0, (JT & m) == 0, m)
        m *= 2

    K, I = _prune_pairs(K, I)
    dirA = lax.broadcasted_iota(jnp.int32, (1, 2 * CHROWS, 1), 1) < CHROWS
    K, I = _merge_tail(K, I, dirA, dirA, K_TOP)
    K, I = _prune_pairs(K, I)
    dirA = jnp.full((1, 1, 1), True)
    K, I = _merge_tail(K, I, dirA, dirA, K_TOP)
    out_ref[...] = I


def _sort(keys):
    nb = keys.shape[0]
    return pl.pallas_call(
        _sort_body,
        in_specs=[pl.BlockSpec((nb, ROWS, LANES), lambda: (0, 0, 0))],
        out_specs=pl.BlockSpec((nb, CHROWS, LANES), lambda: (0, 0, 0)),
        out_shape=jax.ShapeDtypeStruct((nb, CHROWS, LANES), jnp.int32),
        cost_estimate=pl.CostEstimate(
            flops=2_000_000_000, transcendentals=0, bytes_accessed=16 << 20
        ),
    )(keys)


CX = 3


HALF = K_TOP // 2


def _gather_half(row_v, idx_v, out_v, h):
    @plsc.parallel_loop(h * HALF, (h + 1) * HALF, 16, unroll=16)
    def _(i):
        iv = idx_v[pl.ds(i, 16)]
        out_v[pl.ds(i - h * HALF, 16)] = plsc.load_gather(row_v, [iv])


def _sc_gather_body(
    b0, x_hbm, y_hbm, idx_hbm, out_x, out_y, row_v, idx_v, oa_v, ob_v, sem_a, sem_b
):
    cid = lax.axis_index("c")
    sid = lax.axis_index("s")
    wid = sid * 2 + cid
    nw = 8
    b = wid // nw
    j = wid % nw
    pltpu.sync_copy(idx_hbm.at[b], idx_v)

    def do_row(src_row, dst_row, prev):
        pltpu.sync_copy(src_row, row_v)
        if prev is not None:
            prev[0].wait()
            prev[1].wait()
        _gather_half(row_v, idx_v, oa_v, 0)
        da = pltpu.async_copy(oa_v, dst_row.at[pl.ds(0, HALF)], sem_a)
        _gather_half(row_v, idx_v, ob_v, 1)
        db = pltpu.async_copy(ob_v, dst_row.at[pl.ds(HALF, HALF)], sem_b)
        return (da, db)

    prev = None
    for ci in range(C // nw):
        c = j * (C // nw) + ci
        prev = do_row(y_hbm.at[b0 + b, c], out_y.at[b, c], prev)
    prev[0].wait()
    prev[1].wait()

    @pl.when(j < CX)
    def _():
        pltpu.sync_copy(x_hbm.at[b0 + b, j], row_v)
        _gather_half(row_v, idx_v, oa_v, 0)
        _gather_half(row_v, idx_v, ob_v, 1)
        pltpu.sync_copy(oa_v, out_x.at[b, j, pl.ds(0, HALF)])
        pltpu.sync_copy(ob_v, out_x.at[b, j, pl.ds(HALF, HALF)])


@functools.cache
def _make_sc_gather(b0):
    return pl.kernel(
        functools.partial(_sc_gather_body, b0),
        out_type=(
            jax.ShapeDtypeStruct((B // 2, CX, K_TOP), jnp.float32),
            jax.ShapeDtypeStruct((B // 2, C, K_TOP), jnp.float32),
        ),
        mesh=plsc.VectorSubcoreMesh(core_axis_name="c", subcore_axis_name="s"),
        compiler_params=pltpu.CompilerParams(needs_layout_passes=False),
        scratch_types=[
            pltpu.VMEM((N,), jnp.float32),
            pltpu.VMEM((K_TOP,), jnp.int32),
            pltpu.VMEM((HALF,), jnp.float32),
            pltpu.VMEM((HALF,), jnp.float32),
            pltpu.SemaphoreType.DMA,
            pltpu.SemaphoreType.DMA,
        ],
    )


def kernel(x, y):
    keys = _maxkey(y)
    keys = keys.reshape(B, ROWS, LANES)
    h = B // 2
    idx_a = jnp.swapaxes(_sort(keys[:h]), 1, 2).reshape(h, K_TOP)
    gx_a, gy_a = _make_sc_gather(0)(x, y, idx_a)
    idx_b = jnp.swapaxes(_sort(keys[h:]), 1, 2).reshape(h, K_TOP)
    gx_b, gy_b = _make_sc_gather(h)(x, y, idx_b)
    top_k_xyz = jnp.concatenate([gx_a, gx_b], axis=0)
    top_k_points = jnp.concatenate([gy_a, gy_b], axis=0)
    return (top_k_xyz, top_k_points)
